# Initial kernel scaffold; baseline (speedup 1.0000x reference)
import jax
import jax.numpy as jnp
from jax.experimental import pallas as pl


def kernel(input, hash_tables, p, W):
    raise NotImplementedError("write your pallas kernel here")


# speedup vs baseline: 2.4865x; 2.4865x over previous
import functools

import jax
import jax.numpy as jnp
from jax import lax
from jax.experimental import pallas as pl
from jax.experimental.pallas import tpu as pltpu, tpu_sc as plsc

WORD_COUNT = 1000000
NUM_BUCKETS = 100000
EMBED = 64
BATCH = 4096
SEQ = 200

NC = 2
NS = 16
L = 16
NW = NC * NS

N_TOK = BATCH * SEQ
TOK_PER_W = N_TOK // NW
CHUNK = 128
N_CHUNKS = TOK_PER_W // CHUNK
META_W = 16


def _sc_body(tok_hbm, tbl_hbm, w_hbm, out_hbm,
             tok_v, meta_v, idx0_v, idx1_v, p0_v, p1_v,
             w0_v, w1_v, out_v, sem0, sem1):
    wid = lax.axis_index("s") * NC + lax.axis_index("c")
    tile_base = wid * TOK_PER_W
    lane = lax.iota(jnp.int32, L)
    zeros = jnp.zeros((L,), jnp.int32)
    ones = jnp.full((L,), 1, jnp.int32)
    twos = jnp.full((L,), 2, jnp.int32)
    threes = jnp.full((L,), 3, jnp.int32)

    def chunk_body(c, carry):
        base = tile_base + c * CHUNK
        pltpu.sync_copy(tok_hbm.at[pl.ds(base, CHUNK)], tok_v)
        pltpu.async_copy(tbl_hbm.at[tok_v], meta_v, sem0).wait()

        def meta_body(g, _):
            s = g * L
            rows = s + lane
            wv = tok_v[pl.ds(s, L)]
            nz = wv != 0
            b0 = plsc.load_gather(meta_v, [rows, zeros])
            b1 = plsc.load_gather(meta_v, [rows, ones])
            p0b = plsc.load_gather(meta_v, [rows, twos])
            p1b = plsc.load_gather(meta_v, [rows, threes])
            idx0_v[pl.ds(s, L)] = jnp.where(nz, b0, 0)
            idx1_v[pl.ds(s, L)] = jnp.where(nz, b1, 0)
            p0 = plsc.bitcast(p0b, jnp.float32)
            p1 = plsc.bitcast(p1b, jnp.float32)
            p0_v[pl.ds(s, L)] = p0
            p1_v[pl.ds(s, L)] = p1
            plsc.store_scatter(out_v, [rows, jnp.full((L,), EMBED, jnp.int32)], p0)
            plsc.store_scatter(out_v, [rows, jnp.full((L,), EMBED + 1, jnp.int32)], p1)
            return 0

        lax.fori_loop(0, CHUNK // L, meta_body, 0)

        cp0 = pltpu.async_copy(w_hbm.at[idx0_v], w0_v, sem0)
        cp1 = pltpu.async_copy(w_hbm.at[idx1_v], w1_v, sem1)
        cp0.wait()
        cp1.wait()

        def tok_body(i, _):
            p0 = plsc.load_gather(p0_v, [jnp.full((L,), i, jnp.int32)])
            p1 = plsc.load_gather(p1_v, [jnp.full((L,), i, jnp.int32)])
            for k in range(EMBED // L):
                a = w0_v[i, pl.ds(k * L, L)]
                b = w1_v[i, pl.ds(k * L, L)]
                out_v[i, pl.ds(k * L, L)] = a * p0 + b * p1
            return 0

        lax.fori_loop(0, CHUNK, tok_body, 0)

        pltpu.sync_copy(out_v, out_hbm.at[pl.ds(base, CHUNK)])
        return carry

    lax.fori_loop(0, N_CHUNKS, chunk_body, 0)


def kernel(input, hash_tables, p, W):
    tok = input.reshape(N_TOK)
    p_shift = jnp.roll(p, -3, axis=0)
    tbl = jnp.concatenate(
        [hash_tables,
         lax.bitcast_convert_type(p_shift, jnp.int32),
         jnp.zeros((WORD_COUNT, META_W - 4), jnp.int32)], axis=1)

    mesh = plsc.VectorSubcoreMesh(
        core_axis_name="c", subcore_axis_name="s",
        num_cores=NC, num_subcores=NS)
    run = pl.kernel(
        _sc_body,
        out_type=jax.ShapeDtypeStruct((N_TOK, EMBED + 2), jnp.float32),
        mesh=mesh,
        compiler_params=pltpu.CompilerParams(
            needs_layout_passes=False, use_tc_tiling_on_sc=False),
        scratch_types=[
            pltpu.VMEM((CHUNK,), jnp.int32),
            pltpu.VMEM((CHUNK, META_W), jnp.int32),
            pltpu.VMEM((CHUNK,), jnp.int32),
            pltpu.VMEM((CHUNK,), jnp.int32),
            pltpu.VMEM((CHUNK,), jnp.float32),
            pltpu.VMEM((CHUNK,), jnp.float32),
            pltpu.VMEM((CHUNK, EMBED), jnp.float32),
            pltpu.VMEM((CHUNK, EMBED), jnp.float32),
            pltpu.VMEM((CHUNK, EMBED + 2), jnp.float32),
            pltpu.SemaphoreType.DMA,
            pltpu.SemaphoreType.DMA,
        ],
    )
    out = run(tok, tbl, W)
    return out.reshape(BATCH, SEQ, EMBED + 2)
